# initial kernel scaffold (unmeasured)
import jax
import jax.numpy as jnp
from jax import lax
from jax.experimental import pallas as pl
from jax.experimental.pallas import tpu as pltpu


def kernel(
    t,
):
    def body(*refs):
        pass

    out_shape = jax.ShapeDtypeStruct(..., jnp.float32)
    return pl.pallas_call(body, out_shape=out_shape)(...)


# baseline (device time: 55712 ns/iter reference)
import jax
import jax.numpy as jnp
from jax import lax
from jax.experimental import pallas as pl
from jax.experimental.pallas import tpu as pltpu

N_DEV = 8


def kernel(t):
    m, n = t.shape
    assert m % N_DEV == 0

    def body(x_ref, out_ref, rbuf0, rbuf1, rbuf2,
             rs_send, rs_recv, ag_send, ag_recv):
        my_i = lax.axis_index("i")
        lab = my_i ^ ((my_i >> 1) & 1)

        def logical_of(label):
            return label ^ ((label >> 1) & 1)

        partners = [logical_of(lab ^ (1 << r)) for r in range(3)]
        rbufs = [rbuf0, rbuf1, rbuf2]

        barrier_sem = pltpu.get_barrier_semaphore()
        for p in partners:
            pl.semaphore_signal(
                barrier_sem, inc=1,
                device_id=(p,), device_id_type=pl.DeviceIdType.MESH,
            )
        pl.semaphore_wait(barrier_sem, 3)

        out_ref[:, :] = x_ref[:, :]

        base = jnp.int32(0)
        for r in range(3):
            half = m >> (r + 1)
            bit = (lab >> r) & 1
            keep_base = base + bit * half
            send_base = base + (1 - bit) * half
            rdma = pltpu.make_async_remote_copy(
                src_ref=out_ref.at[pl.ds(send_base, half), :],
                dst_ref=rbufs[r].at[:, :],
                send_sem=rs_send.at[r],
                recv_sem=rs_recv.at[r],
                device_id=(partners[r],),
                device_id_type=pl.DeviceIdType.MESH,
            )
            rdma.start()
            rdma.wait()
            out_ref[pl.ds(keep_base, half), :] = (
                out_ref[pl.ds(keep_base, half), :] + rbufs[r][:, :]
            )
            base = keep_base

        chunk = m >> 3
        s = out_ref[pl.ds(base, chunk), :]
        relu = jnp.maximum(s, 0.0)
        out_ref[pl.ds(base, chunk), :] = (
            jnp.tanh(s) * s * s + relu * relu * relu
        )

        for r in (2, 1, 0):
            size = m >> (r + 1)
            bit = (lab >> r) & 1
            rdma = pltpu.make_async_remote_copy(
                src_ref=out_ref.at[pl.ds(base, size), :],
                dst_ref=out_ref.at[pl.ds(base, size), :],
                send_sem=ag_send.at[r],
                recv_sem=ag_recv.at[r],
                device_id=(partners[r],),
                device_id_type=pl.DeviceIdType.MESH,
            )
            rdma.start()
            rdma.wait()
            base = base - bit * size

    return pl.pallas_call(
        body,
        out_shape=jax.ShapeDtypeStruct((m, n), jnp.float32),
        in_specs=[pl.BlockSpec(memory_space=pltpu.VMEM)],
        out_specs=pl.BlockSpec(memory_space=pltpu.VMEM),
        scratch_shapes=[
            pltpu.VMEM((m // 2, n), jnp.float32),
            pltpu.VMEM((m // 4, n), jnp.float32),
            pltpu.VMEM((m // 8, n), jnp.float32),
            pltpu.SemaphoreType.DMA((3,)),
            pltpu.SemaphoreType.DMA((3,)),
            pltpu.SemaphoreType.DMA((3,)),
            pltpu.SemaphoreType.DMA((3,)),
        ],
        compiler_params=pltpu.CompilerParams(collective_id=0),
    )(t)


# device time: 32370 ns/iter; 1.7211x vs baseline; 1.7211x over previous
import jax
import jax.numpy as jnp
from jax import lax
from jax.experimental import pallas as pl
from jax.experimental.pallas import tpu as pltpu

N_DEV = 8

PARTS = (
    (0, 384, (0, 1, 2)),
    (384, 384, (1, 2, 0)),
    (768, 256, (2, 0, 1)),
)


def kernel(t):
    m, n = t.shape
    assert m == sum(p[1] for p in PARTS)

    rb_off = []
    off = 0
    for _, rows, _ in PARTS:
        offs = []
        for s in range(3):
            offs.append(off)
            off += rows >> (s + 1)
        rb_off.append(offs)
    rb_rows = off

    def body(x_ref, out_ref, rbuf, rs_send, rs_recv, ag_send, ag_recv):
        my_i = lax.axis_index("i")
        lab = my_i ^ ((my_i >> 1) & 1)

        def logical_of(label):
            return label ^ ((label >> 1) & 1)

        nbr = [logical_of(lab ^ (1 << a)) for a in range(3)]

        barrier_sem = pltpu.get_barrier_semaphore()
        for p in nbr:
            pl.semaphore_signal(
                barrier_sem, inc=1,
                device_id=(p,), device_id_type=pl.DeviceIdType.MESH,
            )
        pl.semaphore_wait(barrier_sem, 3)

        out_ref[:, :] = x_ref[:, :]

        base = [jnp.int32(p[0]) for p in PARTS]

        for s in range(3):
            rdmas = []
            keeps = []
            for k, (_, rows, order) in enumerate(PARTS):
                a = order[s]
                bit = (lab >> a) & 1
                half = rows >> (s + 1)
                keep_base = base[k] + bit * half
                send_base = base[k] + (1 - bit) * half
                rdma = pltpu.make_async_remote_copy(
                    src_ref=out_ref.at[pl.ds(send_base, half), :],
                    dst_ref=rbuf.at[pl.ds(rb_off[k][s], half), :],
                    send_sem=rs_send.at[3 * k + s],
                    recv_sem=rs_recv.at[3 * k + s],
                    device_id=(nbr[a],),
                    device_id_type=pl.DeviceIdType.MESH,
                )
                rdma.start()
                rdmas.append(rdma)
                keeps.append((keep_base, half))
                base[k] = keep_base
            for k, rdma in enumerate(rdmas):
                rdma.wait()
                kb, half = keeps[k]
                out_ref[pl.ds(kb, half), :] = (
                    out_ref[pl.ds(kb, half), :]
                    + rbuf[pl.ds(rb_off[k][s], half), :]
                )

        for k, (_, rows, _) in enumerate(PARTS):
            chunk = rows >> 3
            s_ = out_ref[pl.ds(base[k], chunk), :]
            relu = jnp.maximum(s_, 0.0)
            out_ref[pl.ds(base[k], chunk), :] = (
                jnp.tanh(s_) * s_ * s_ + relu * relu * relu
            )

        for s in (2, 1, 0):
            rdmas = []
            for k, (_, rows, order) in enumerate(PARTS):
                a = order[s]
                size = rows >> (s + 1)
                rdma = pltpu.make_async_remote_copy(
                    src_ref=out_ref.at[pl.ds(base[k], size), :],
                    dst_ref=out_ref.at[pl.ds(base[k], size), :],
                    send_sem=ag_send.at[3 * k + s],
                    recv_sem=ag_recv.at[3 * k + s],
                    device_id=(nbr[a],),
                    device_id_type=pl.DeviceIdType.MESH,
                )
                rdma.start()
                rdmas.append(rdma)
            for k, (_, rows, order) in enumerate(PARTS):
                rdmas[k].wait()
                a = order[s]
                bit = (lab >> a) & 1
                base[k] = base[k] - bit * (rows >> (s + 1))

    return pl.pallas_call(
        body,
        out_shape=jax.ShapeDtypeStruct((m, n), jnp.float32),
        in_specs=[pl.BlockSpec(memory_space=pltpu.VMEM)],
        out_specs=pl.BlockSpec(memory_space=pltpu.VMEM),
        scratch_shapes=[
            pltpu.VMEM((rb_rows, n), jnp.float32),
            pltpu.SemaphoreType.DMA((9,)),
            pltpu.SemaphoreType.DMA((9,)),
            pltpu.SemaphoreType.DMA((9,)),
            pltpu.SemaphoreType.DMA((9,)),
        ],
        compiler_params=pltpu.CompilerParams(collective_id=0),
    )(t)


# device time: 32002 ns/iter; 1.7409x vs baseline; 1.0115x over previous
import jax
import jax.numpy as jnp
from jax import lax
from jax.experimental import pallas as pl
from jax.experimental.pallas import tpu as pltpu

N_DEV = 8

PARTS = (
    (0, 384, (0, 1, 2)),
    (384, 384, (1, 2, 0)),
    (768, 256, (2, 0, 1)),
)


def kernel(t):
    m, n = t.shape
    assert m == sum(p[1] for p in PARTS)

    rb_off = []
    off = 0
    for _, rows, _ in PARTS:
        offs = []
        for s in range(3):
            offs.append(off)
            off += rows >> (s + 1)
        rb_off.append(offs)
    rb_rows = off

    def body(x_ref, out_ref, rbuf, rs_send, rs_recv, ag_send, ag_recv):
        my_i = lax.axis_index("i")
        lab = my_i ^ ((my_i >> 1) & 1)

        def logical_of(label):
            return label ^ ((label >> 1) & 1)

        nbr = [logical_of(lab ^ (1 << a)) for a in range(3)]

        barrier_sem = pltpu.get_barrier_semaphore()
        for p in nbr:
            pl.semaphore_signal(
                barrier_sem, inc=1,
                device_id=(p,), device_id_type=pl.DeviceIdType.MESH,
            )
        pl.semaphore_wait(barrier_sem, 3)

        base = [jnp.int32(p[0]) for p in PARTS]
        keep = [None] * 3
        rs_rdma = [None] * 3
        ag_rdma = [None] * 3

        def start_rs(k, s):
            _, rows, order = PARTS[k]
            a = order[s]
            bit = (lab >> a) & 1
            half = rows >> (s + 1)
            keep_base = base[k] + bit * half
            send_base = base[k] + (1 - bit) * half
            src = x_ref if s == 0 else out_ref
            rdma = pltpu.make_async_remote_copy(
                src_ref=src.at[pl.ds(send_base, half), :],
                dst_ref=rbuf.at[pl.ds(rb_off[k][s], half), :],
                send_sem=rs_send.at[3 * k + s],
                recv_sem=rs_recv.at[3 * k + s],
                device_id=(nbr[a],),
                device_id_type=pl.DeviceIdType.MESH,
            )
            rdma.start()
            rs_rdma[k] = rdma
            keep[k] = (keep_base, half)
            base[k] = keep_base

        def start_ag(k, s):
            _, rows, order = PARTS[k]
            size = rows >> (s + 1)
            rdma = pltpu.make_async_remote_copy(
                src_ref=out_ref.at[pl.ds(base[k], size), :],
                dst_ref=out_ref.at[pl.ds(base[k], size), :],
                send_sem=ag_send.at[3 * k + s],
                recv_sem=ag_recv.at[3 * k + s],
                device_id=(nbr[order[s]],),
                device_id_type=pl.DeviceIdType.MESH,
            )
            rdma.start()
            ag_rdma[k] = rdma

        for k in range(3):
            start_rs(k, 0)

        for s in range(3):
            for k in range(3):
                rs_rdma[k].wait()
                kb, half = keep[k]
                prev = x_ref if s == 0 else out_ref
                out_ref[pl.ds(kb, half), :] = (
                    prev[pl.ds(kb, half), :]
                    + rbuf[pl.ds(rb_off[k][s], half), :]
                )
                if s < 2:
                    start_rs(k, s + 1)
                else:
                    chunk = PARTS[k][1] >> 3
                    s_ = out_ref[pl.ds(base[k], chunk), :]
                    relu = jnp.maximum(s_, 0.0)
                    out_ref[pl.ds(base[k], chunk), :] = (
                        jnp.tanh(s_) * s_ * s_ + relu * relu * relu
                    )
                    start_ag(k, 2)

        for s in (2, 1, 0):
            for k in range(3):
                ag_rdma[k].wait()
                _, rows, order = PARTS[k]
                bit = (lab >> order[s]) & 1
                base[k] = base[k] - bit * (rows >> (s + 1))
                if s > 0:
                    start_ag(k, s - 1)

    return pl.pallas_call(
        body,
        out_shape=jax.ShapeDtypeStruct((m, n), jnp.float32),
        in_specs=[pl.BlockSpec(memory_space=pltpu.VMEM)],
        out_specs=pl.BlockSpec(memory_space=pltpu.VMEM),
        scratch_shapes=[
            pltpu.VMEM((rb_rows, n), jnp.float32),
            pltpu.SemaphoreType.DMA((9,)),
            pltpu.SemaphoreType.DMA((9,)),
            pltpu.SemaphoreType.DMA((9,)),
            pltpu.SemaphoreType.DMA((9,)),
        ],
        compiler_params=pltpu.CompilerParams(collective_id=0),
    )(t)


# device time: 29353 ns/iter; 1.8980x vs baseline; 1.0902x over previous
import jax
import jax.numpy as jnp
from jax import lax
from jax.experimental import pallas as pl
from jax.experimental.pallas import tpu as pltpu

N_DEV = 8

PARTS = (
    (0, 192, (0, 1, 2)),
    (192, 192, (0, 1, 2)),
    (384, 192, (1, 2, 0)),
    (576, 192, (1, 2, 0)),
    (768, 128, (2, 0, 1)),
    (896, 128, (2, 0, 1)),
)
N_PARTS = len(PARTS)


def kernel(t):
    m, n = t.shape
    assert m == sum(p[1] for p in PARTS)

    rb_off = []
    off = 0
    for _, rows, _ in PARTS:
        offs = []
        for s in range(3):
            offs.append(off)
            off += rows >> (s + 1)
        rb_off.append(offs)
    rb_rows = off

    def body(x_ref, out_ref, rbuf, rs_send, rs_recv, ag_send, ag_recv):
        my_i = lax.axis_index("i")
        lab = my_i ^ ((my_i >> 1) & 1)

        def logical_of(label):
            return label ^ ((label >> 1) & 1)

        nbr = [logical_of(lab ^ (1 << a)) for a in range(3)]

        barrier_sem = pltpu.get_barrier_semaphore()
        for p in nbr:
            pl.semaphore_signal(
                barrier_sem, inc=1,
                device_id=(p,), device_id_type=pl.DeviceIdType.MESH,
            )
        pl.semaphore_wait(barrier_sem, 3)

        base = [jnp.int32(p[0]) for p in PARTS]
        keep = [None] * N_PARTS
        rs_rdma = [None] * N_PARTS
        ag_rdma = [None] * N_PARTS

        def start_rs(k, s):
            _, rows, order = PARTS[k]
            a = order[s]
            bit = (lab >> a) & 1
            half = rows >> (s + 1)
            keep_base = base[k] + bit * half
            send_base = base[k] + (1 - bit) * half
            src = x_ref if s == 0 else out_ref
            rdma = pltpu.make_async_remote_copy(
                src_ref=src.at[pl.ds(send_base, half), :],
                dst_ref=rbuf.at[pl.ds(rb_off[k][s], half), :],
                send_sem=rs_send.at[3 * k + s],
                recv_sem=rs_recv.at[3 * k + s],
                device_id=(nbr[a],),
                device_id_type=pl.DeviceIdType.MESH,
            )
            rdma.start()
            rs_rdma[k] = rdma
            keep[k] = (keep_base, half)
            base[k] = keep_base

        def start_ag(k, s):
            _, rows, order = PARTS[k]
            size = rows >> (s + 1)
            rdma = pltpu.make_async_remote_copy(
                src_ref=out_ref.at[pl.ds(base[k], size), :],
                dst_ref=out_ref.at[pl.ds(base[k], size), :],
                send_sem=ag_send.at[3 * k + s],
                recv_sem=ag_recv.at[3 * k + s],
                device_id=(nbr[order[s]],),
                device_id_type=pl.DeviceIdType.MESH,
            )
            rdma.start()
            ag_rdma[k] = rdma

        for k in range(N_PARTS):
            start_rs(k, 0)

        for s in range(3):
            for k in range(N_PARTS):
                rs_rdma[k].wait()
                kb, half = keep[k]
                prev = x_ref if s == 0 else out_ref
                out_ref[pl.ds(kb, half), :] = (
                    prev[pl.ds(kb, half), :]
                    + rbuf[pl.ds(rb_off[k][s], half), :]
                )
                if s < 2:
                    start_rs(k, s + 1)
                else:
                    chunk = PARTS[k][1] >> 3
                    s_ = out_ref[pl.ds(base[k], chunk), :]
                    relu = jnp.maximum(s_, 0.0)
                    out_ref[pl.ds(base[k], chunk), :] = (
                        jnp.tanh(s_) * s_ * s_ + relu * relu * relu
                    )
                    start_ag(k, 2)

        for s in (2, 1, 0):
            for k in range(N_PARTS):
                ag_rdma[k].wait()
                _, rows, order = PARTS[k]
                bit = (lab >> order[s]) & 1
                base[k] = base[k] - bit * (rows >> (s + 1))
                if s > 0:
                    start_ag(k, s - 1)

    return pl.pallas_call(
        body,
        out_shape=jax.ShapeDtypeStruct((m, n), jnp.float32),
        in_specs=[pl.BlockSpec(memory_space=pltpu.VMEM)],
        out_specs=pl.BlockSpec(memory_space=pltpu.VMEM),
        scratch_shapes=[
            pltpu.VMEM((rb_rows, n), jnp.float32),
            pltpu.SemaphoreType.DMA((3 * N_PARTS,)),
            pltpu.SemaphoreType.DMA((3 * N_PARTS,)),
            pltpu.SemaphoreType.DMA((3 * N_PARTS,)),
            pltpu.SemaphoreType.DMA((3 * N_PARTS,)),
        ],
        compiler_params=pltpu.CompilerParams(collective_id=0),
    )(t)


# device time: 22259 ns/iter; 2.5029x vs baseline; 1.3187x over previous
import jax
import jax.numpy as jnp
from jax import lax
from jax.experimental import pallas as pl
from jax.experimental.pallas import tpu as pltpu

N_DEV = 8

PARTS = (
    (0, 128, (0, 1, 2)),
    (128, 128, (0, 1, 2)),
    (256, 128, (0, 1, 2)),
    (384, 128, (1, 2, 0)),
    (512, 128, (1, 2, 0)),
    (640, 128, (1, 2, 0)),
    (768, 128, (2, 0, 1)),
    (896, 128, (2, 0, 1)),
)
N_PARTS = len(PARTS)


def kernel(t):
    m, n = t.shape
    assert m == sum(p[1] for p in PARTS)

    rb_off = []
    off = 0
    for _, rows, _ in PARTS:
        offs = []
        for s in range(3):
            offs.append(off)
            off += rows >> (s + 1)
        rb_off.append(offs)
    rb_rows = off

    def body(x_ref, out_ref, xbf, rbuf, rs_send, rs_recv, ag_send, ag_recv):
        my_i = lax.axis_index("i")
        lab = my_i ^ ((my_i >> 1) & 1)

        def logical_of(label):
            return label ^ ((label >> 1) & 1)

        nbr = [logical_of(lab ^ (1 << a)) for a in range(3)]

        barrier_sem = pltpu.get_barrier_semaphore()
        for p in nbr:
            pl.semaphore_signal(
                barrier_sem, inc=1,
                device_id=(p,), device_id_type=pl.DeviceIdType.MESH,
            )
        pl.semaphore_wait(barrier_sem, 3)

        base = [jnp.int32(p[0]) for p in PARTS]
        keep = [None] * N_PARTS
        rs_rdma = [None] * N_PARTS
        ag_rdma = [None] * N_PARTS

        def start_rs(k, s):
            _, rows, order = PARTS[k]
            a = order[s]
            bit = (lab >> a) & 1
            half = rows >> (s + 1)
            keep_base = base[k] + bit * half
            send_base = base[k] + (1 - bit) * half
            rdma = pltpu.make_async_remote_copy(
                src_ref=xbf.at[pl.ds(send_base, half), :],
                dst_ref=rbuf.at[pl.ds(rb_off[k][s], half), :],
                send_sem=rs_send.at[3 * k + s],
                recv_sem=rs_recv.at[3 * k + s],
                device_id=(nbr[a],),
                device_id_type=pl.DeviceIdType.MESH,
            )
            rdma.start()
            rs_rdma[k] = rdma
            keep[k] = (keep_base, half)
            base[k] = keep_base

        def start_ag(k, s):
            _, rows, order = PARTS[k]
            size = rows >> (s + 1)
            rdma = pltpu.make_async_remote_copy(
                src_ref=xbf.at[pl.ds(base[k], size), :],
                dst_ref=xbf.at[pl.ds(base[k], size), :],
                send_sem=ag_send.at[3 * k + s],
                recv_sem=ag_recv.at[3 * k + s],
                device_id=(nbr[order[s]],),
                device_id_type=pl.DeviceIdType.MESH,
            )
            rdma.start()
            ag_rdma[k] = rdma

        for k, (pb, rows, _) in enumerate(PARTS):
            xbf[pl.ds(pb, rows), :] = x_ref[pl.ds(pb, rows), :].astype(
                jnp.bfloat16
            )
            start_rs(k, 0)

        for s in range(3):
            for k in range(N_PARTS):
                rs_rdma[k].wait()
                kb, half = keep[k]
                prev = x_ref if s == 0 else out_ref
                acc = (
                    prev[pl.ds(kb, half), :]
                    + rbuf[pl.ds(rb_off[k][s], half), :].astype(jnp.float32)
                )
                out_ref[pl.ds(kb, half), :] = acc
                xbf[pl.ds(kb, half), :] = acc.astype(jnp.bfloat16)
                if s < 2:
                    start_rs(k, s + 1)
                else:
                    chunk = PARTS[k][1] >> 3
                    s_ = out_ref[pl.ds(base[k], chunk), :]
                    relu = jnp.maximum(s_, 0.0)
                    xbf[pl.ds(base[k], chunk), :] = (
                        jnp.tanh(s_) * s_ * s_ + relu * relu * relu
                    ).astype(jnp.bfloat16)
                    start_ag(k, 2)

        for s in (2, 1, 0):
            for k in range(N_PARTS):
                ag_rdma[k].wait()
                _, rows, order = PARTS[k]
                bit = (lab >> order[s]) & 1
                base[k] = base[k] - bit * (rows >> (s + 1))
                if s > 0:
                    start_ag(k, s - 1)
                else:
                    pb = PARTS[k][0]
                    out_ref[pl.ds(pb, rows), :] = xbf[
                        pl.ds(pb, rows), :
                    ].astype(jnp.float32)

    return pl.pallas_call(
        body,
        out_shape=jax.ShapeDtypeStruct((m, n), jnp.float32),
        in_specs=[pl.BlockSpec(memory_space=pltpu.VMEM)],
        out_specs=pl.BlockSpec(memory_space=pltpu.VMEM),
        scratch_shapes=[
            pltpu.VMEM((m, n), jnp.bfloat16),
            pltpu.VMEM((rb_rows, n), jnp.bfloat16),
            pltpu.SemaphoreType.DMA((3 * N_PARTS,)),
            pltpu.SemaphoreType.DMA((3 * N_PARTS,)),
            pltpu.SemaphoreType.DMA((3 * N_PARTS,)),
            pltpu.SemaphoreType.DMA((3 * N_PARTS,)),
        ],
        compiler_params=pltpu.CompilerParams(collective_id=0),
    )(t)


# device time: 17839 ns/iter; 3.1230x vs baseline; 1.2478x over previous
import jax
import jax.numpy as jnp
from jax import lax
from jax.experimental import pallas as pl
from jax.experimental.pallas import tpu as pltpu

N_DEV = 8

PARTS = tuple(
    [(64 * i, 64, (0, 1, 2)) for i in range(6)]
    + [(384 + 64 * i, 64, (1, 2, 0)) for i in range(5)]
    + [(704 + 64 * i, 64, (2, 0, 1)) for i in range(5)]
)
N_PARTS = len(PARTS)
K_ORDER = (0, 6, 11, 1, 7, 12, 2, 8, 13, 3, 9, 14, 4, 10, 15, 5)


def kernel(t):
    m, n = t.shape
    assert m == sum(p[1] for p in PARTS)

    rb_off = []
    off = 0
    for _, rows, _ in PARTS:
        offs = []
        for s in range(3):
            offs.append(off)
            off += rows >> 1
        rb_off.append(offs)
    rb_rows = off

    def body(x_ref, out_ref, xbf, rbuf, rs_send, rs_recv, ag_send, ag_recv):
        my_i = lax.axis_index("i")
        lab = my_i ^ ((my_i >> 1) & 1)

        def logical_of(label):
            return label ^ ((label >> 1) & 1)

        nbr = [logical_of(lab ^ (1 << a)) for a in range(3)]

        barrier_sem = pltpu.get_barrier_semaphore()
        for p in nbr:
            pl.semaphore_signal(
                barrier_sem, inc=1,
                device_id=(p,), device_id_type=pl.DeviceIdType.MESH,
            )
        for pb, rows, _ in PARTS:
            xbf[pl.ds(pb, rows), :] = x_ref[pl.ds(pb, rows), :].astype(
                jnp.bfloat16
            )
        pl.semaphore_wait(barrier_sem, 3)

        base = [jnp.int32(p[0]) for p in PARTS]
        keep = [None] * N_PARTS
        rs_rdma = [None] * N_PARTS
        ag_rdma = [None] * N_PARTS

        def start_rs(k, s):
            _, rows, order = PARTS[k]
            a = order[s]
            half = rows >> 1
            if s == 0:
                bit = (lab >> a) & 1
                keep_base = base[k] + bit * half
                send_base = base[k] + (1 - bit) * half
            else:
                keep_base = base[k]
                send_base = base[k]
            rdma = pltpu.make_async_remote_copy(
                src_ref=xbf.at[pl.ds(send_base, half), :],
                dst_ref=rbuf.at[pl.ds(rb_off[k][s], half), :],
                send_sem=rs_send.at[3 * k + s],
                recv_sem=rs_recv.at[3 * k + s],
                device_id=(nbr[a],),
                device_id_type=pl.DeviceIdType.MESH,
            )
            rdma.start()
            rs_rdma[k] = rdma
            keep[k] = (keep_base, half)
            base[k] = keep_base

        def start_ag(k):
            _, rows, order = PARTS[k]
            size = rows >> 1
            rdma = pltpu.make_async_remote_copy(
                src_ref=xbf.at[pl.ds(base[k], size), :],
                dst_ref=xbf.at[pl.ds(base[k], size), :],
                send_sem=ag_send.at[k],
                recv_sem=ag_recv.at[k],
                device_id=(nbr[order[0]],),
                device_id_type=pl.DeviceIdType.MESH,
            )
            rdma.start()
            ag_rdma[k] = rdma

        for k in K_ORDER:
            start_rs(k, 0)

        for s in range(3):
            for k in K_ORDER:
                rs_rdma[k].wait()
                kb, half = keep[k]
                prev = x_ref if s == 0 else out_ref
                acc = (
                    prev[pl.ds(kb, half), :]
                    + rbuf[pl.ds(rb_off[k][s], half), :].astype(jnp.float32)
                )
                if s < 2:
                    xbf[pl.ds(kb, half), :] = acc.astype(jnp.bfloat16)
                    start_rs(k, s + 1)
                    out_ref[pl.ds(kb, half), :] = acc
                else:
                    relu = jnp.maximum(acc, 0.0)
                    xbf[pl.ds(kb, half), :] = (
                        jnp.tanh(acc) * acc * acc + relu * relu * relu
                    ).astype(jnp.bfloat16)
                    start_ag(k)

        for k in K_ORDER:
            ag_rdma[k].wait()
            pb, rows, _ = PARTS[k]
            out_ref[pl.ds(pb, rows), :] = xbf[
                pl.ds(pb, rows), :
            ].astype(jnp.float32)

    return pl.pallas_call(
        body,
        out_shape=jax.ShapeDtypeStruct((m, n), jnp.float32),
        in_specs=[pl.BlockSpec(memory_space=pltpu.VMEM)],
        out_specs=pl.BlockSpec(memory_space=pltpu.VMEM),
        scratch_shapes=[
            pltpu.VMEM((m, n), jnp.bfloat16),
            pltpu.VMEM((rb_rows, n), jnp.bfloat16),
            pltpu.SemaphoreType.DMA((3 * N_PARTS,)),
            pltpu.SemaphoreType.DMA((3 * N_PARTS,)),
            pltpu.SemaphoreType.DMA((N_PARTS,)),
            pltpu.SemaphoreType.DMA((N_PARTS,)),
        ],
        compiler_params=pltpu.CompilerParams(collective_id=0),
    )(t)
